# baseline (speedup 1.0000x reference)
import functools

import jax
import jax.numpy as jnp
from jax import lax
from jax.experimental import pallas as pl
from jax.experimental.pallas import tpu as pltpu
from jax.experimental.pallas import tpu_sc as plsc

NUM_NODES = 1000000
DIM = 128
EPS = 1e-05
B_TOTAL = 16384 * 50
NW = 32
B_PER_W = B_TOTAL // NW
CHUNK = 128
N_CHUNKS = B_PER_W // CHUNK

MAX_NORM = 1.0 - EPS
MAX_NORM2 = MAX_NORM * MAX_NORM


def _rsqrt(x):
    bits = lax.bitcast_convert_type(x, jnp.int32)
    seed = lax.bitcast_convert_type(
        jnp.int32(0x5F3759DF) - lax.shift_right_arithmetic(bits, 1),
        jnp.float32)
    y = seed
    for _ in range(3):
        y = y * (jnp.float32(1.5) - jnp.float32(0.5) * x * y * y)
    return y


def _body(idx_hbm, table_hbm, out_hbm, idx_v, rows_v, sem):
    wid = lax.axis_index("s") * 2 + lax.axis_index("c")
    base = wid * B_PER_W

    def chunk_body(g, _):
        row0 = base + g * CHUNK
        pltpu.sync_copy(idx_hbm.at[pl.ds(row0, CHUNK)], idx_v)
        pltpu.async_copy(table_hbm.at[idx_v], rows_v, sem).wait()

        def row_body(i, _):
            vs = [rows_v[i, pl.ds(16 * j, 16)] for j in range(8)]
            acc = vs[0] * vs[0]
            for j in range(1, 8):
                acc = acc + vs[j] * vs[j]
            norm2 = jnp.sum(acc)
            scale = jnp.where(norm2 > jnp.float32(MAX_NORM2),
                              jnp.float32(MAX_NORM) * _rsqrt(norm2),
                              jnp.float32(1.0))
            for j in range(8):
                rows_v[i, pl.ds(16 * j, 16)] = vs[j] * scale
            return _

        lax.fori_loop(0, CHUNK, row_body, None)
        pltpu.sync_copy(rows_v, out_hbm.at[pl.ds(row0, CHUNK)])
        return _

    lax.fori_loop(0, N_CHUNKS, chunk_body, None)


@jax.jit
def _impl(idx, embeddings):
    mesh = plsc.VectorSubcoreMesh(core_axis_name="c", subcore_axis_name="s")
    f = pl.kernel(
        _body,
        mesh=mesh,
        out_type=jax.ShapeDtypeStruct((B_TOTAL, DIM), jnp.float32),
        scratch_types=[
            pltpu.VMEM((CHUNK,), jnp.int32),
            pltpu.VMEM((CHUNK, DIM), jnp.float32),
            pltpu.SemaphoreType.DMA,
        ],
        compiler_params=pltpu.CompilerParams(needs_layout_passes=False),
    )
    return f(idx, embeddings)


def kernel(indices, embeddings):
    idx = indices.reshape(-1).astype(jnp.int32)
    out = _impl(idx, embeddings)
    return out.reshape(indices.shape[0], indices.shape[1], DIM)


# speedup vs baseline: 1.8840x; 1.8840x over previous
import jax
import jax.numpy as jnp
from jax import lax
from jax.experimental import pallas as pl
from jax.experimental.pallas import tpu as pltpu
from jax.experimental.pallas import tpu_sc as plsc

NUM_NODES = 1000000
DIM = 128
EPS = 1e-05
B_TOTAL = 16384 * 50
NW = 32
B_PER_W = B_TOTAL // NW
CHUNK = 128
N_CHUNKS = B_PER_W // CHUNK
NBUF = 4

MAX_NORM = 1.0 - EPS
MAX_NORM2 = MAX_NORM * MAX_NORM


def _rsqrt(x):
    bits = lax.bitcast_convert_type(x, jnp.int32)
    y = lax.bitcast_convert_type(
        jnp.int32(0x5F3759DF) - lax.shift_right_arithmetic(bits, 1),
        jnp.float32)
    for _ in range(3):
        y = y * (jnp.float32(1.5) - jnp.float32(0.5) * x * y * y)
    return y


def _compute_chunk(rows, nrm_v, lane):
    def group_body(g, _):
        for r in range(16):
            row = g * 16 + r
            vs = [rows[row, pl.ds(16 * j, 16)] for j in range(8)]
            sq = [v * v for v in vs]
            s0 = (sq[0] + sq[1]) + (sq[2] + sq[3])
            s1 = (sq[4] + sq[5]) + (sq[6] + sq[7])
            nrm_v[r, :] = s0 + s1
        tot = plsc.load_gather(nrm_v, [lane, jnp.full((16,), 0, jnp.int32)])
        for l in range(1, 16):
            tot = tot + plsc.load_gather(
                nrm_v, [lane, jnp.full((16,), l, jnp.int32)])
        over = tot > jnp.float32(MAX_NORM2)

        @pl.when(jnp.any(over))
        def _rescale():
            y = _rsqrt(tot)
            sel = jnp.where(over, jnp.float32(MAX_NORM) * y,
                            jnp.float32(1.0))
            for r in range(16):
                row = g * 16 + r
                s = sel[r]
                for j in range(8):
                    rows[row, pl.ds(16 * j, 16)] = (
                        rows[row, pl.ds(16 * j, 16)] * s)

        return _

    lax.fori_loop(0, CHUNK // 16, group_body, None)


def _body(idx_hbm, table_hbm, out_hbm, idx_v, b0, b1, b2, b3,
          g0, g1, g2, g3, w0, w1, w2, w3, nrm_v):
    bufs = (b0, b1, b2, b3)
    gsem = (g0, g1, g2, g3)
    wsem = (w0, w1, w2, w3)
    wid = lax.axis_index("s") * 2 + lax.axis_index("c")
    cbase = wid * N_CHUNKS
    rbase = wid * B_PER_W
    lane = lax.iota(jnp.int32, 16)

    pltpu.sync_copy(idx_hbm.at[pl.ds(cbase, N_CHUNKS)], idx_v)

    pltpu.async_copy(table_hbm.at[idx_v.at[0]], b0, g0)
    pltpu.async_copy(table_hbm.at[idx_v.at[1]], b1, g1)

    def quad_body(i, _):
        for b in range(NBUF):
            c = i * NBUF + b
            bn = (b + 2) % NBUF
            pltpu.make_async_copy(table_hbm.at[idx_v.at[c]], bufs[b],
                                  gsem[b]).wait()
            _compute_chunk(bufs[b], nrm_v, lane)

            @pl.when(c >= 2)
            def _retire():
                pltpu.make_async_copy(
                    bufs[bn],
                    out_hbm.at[pl.ds(rbase, CHUNK)], wsem[bn]).wait()

            @pl.when(c + 2 < N_CHUNKS)
            def _prefetch():
                pltpu.async_copy(table_hbm.at[idx_v.at[c + 2]], bufs[bn],
                                 gsem[bn])

            pltpu.async_copy(
                bufs[b], out_hbm.at[pl.ds(rbase + c * CHUNK, CHUNK)], wsem[b])
        return _

    lax.fori_loop(0, N_CHUNKS // NBUF, quad_body, None)

    for b in ((N_CHUNKS - 2) % NBUF, (N_CHUNKS - 1) % NBUF):
        pltpu.make_async_copy(bufs[b], out_hbm.at[pl.ds(rbase, CHUNK)],
                              wsem[b]).wait()


@jax.jit
def _impl(idx, embeddings):
    mesh = plsc.VectorSubcoreMesh(core_axis_name="c", subcore_axis_name="s")
    f = pl.kernel(
        _body,
        mesh=mesh,
        out_type=jax.ShapeDtypeStruct((B_TOTAL, DIM), jnp.float32),
        scratch_types=[
            pltpu.VMEM((NW * N_CHUNKS // NW, CHUNK), jnp.int32),
            pltpu.VMEM((CHUNK, DIM), jnp.float32),
            pltpu.VMEM((CHUNK, DIM), jnp.float32),
            pltpu.VMEM((CHUNK, DIM), jnp.float32),
            pltpu.VMEM((CHUNK, DIM), jnp.float32),
            pltpu.SemaphoreType.DMA,
            pltpu.SemaphoreType.DMA,
            pltpu.SemaphoreType.DMA,
            pltpu.SemaphoreType.DMA,
            pltpu.SemaphoreType.DMA,
            pltpu.SemaphoreType.DMA,
            pltpu.SemaphoreType.DMA,
            pltpu.SemaphoreType.DMA,
            pltpu.VMEM((16, 16), jnp.float32),
        ],
        compiler_params=pltpu.CompilerParams(needs_layout_passes=False),
    )
    return f(idx.reshape(NW * N_CHUNKS, CHUNK), embeddings)


def kernel(indices, embeddings):
    idx = indices.reshape(-1).astype(jnp.int32)
    out = _impl(idx, embeddings)
    return out.reshape(indices.shape[0], indices.shape[1], DIM)


# speedup vs baseline: 2.8565x; 1.5162x over previous
import jax
import jax.numpy as jnp
from jax import lax
from jax.experimental import pallas as pl
from jax.experimental.pallas import tpu as pltpu
from jax.experimental.pallas import tpu_sc as plsc

NUM_NODES = 1000000
DIM = 128
EPS = 1e-05
N_ROWS = 16384
N_COLS = 50
NW = 32
ROWS_PER_W = N_ROWS // NW
B_PER_W = ROWS_PER_W * N_COLS
CHUNK = 2 * N_COLS
N_CHUNKS = B_PER_W // CHUNK
NBUF = 4
NFULL = CHUNK // 16
NTAIL = CHUNK - NFULL * 16

MAX_NORM = 1.0 - EPS
MAX_NORM2 = MAX_NORM * MAX_NORM


def _rsqrt(x):
    bits = lax.bitcast_convert_type(x, jnp.int32)
    y = lax.bitcast_convert_type(
        jnp.int32(0x5F3759DF) - lax.shift_right_arithmetic(bits, 1),
        jnp.float32)
    for _ in range(3):
        y = y * (jnp.float32(1.5) - jnp.float32(0.5) * x * y * y)
    return y


def _norm_group(rows, nrm_v, lane, g0, nrows):
    for r in range(nrows):
        row = g0 + r
        vs = [rows[row, pl.ds(16 * j, 16)] for j in range(8)]
        sq = [v * v for v in vs]
        s0 = (sq[0] + sq[1]) + (sq[2] + sq[3])
        s1 = (sq[4] + sq[5]) + (sq[6] + sq[7])
        nrm_v[r, :] = s0 + s1
    tot = plsc.load_gather(nrm_v, [lane, jnp.full((16,), 0, jnp.int32)])
    for l in range(1, 16):
        tot = tot + plsc.load_gather(
            nrm_v, [lane, jnp.full((16,), l, jnp.int32)])
    over = tot > jnp.float32(MAX_NORM2)
    if nrows < 16:
        over = jnp.logical_and(over, lane < nrows)

    @pl.when(jnp.any(over))
    def _rescale():
        y = _rsqrt(tot)
        sel = jnp.where(over, jnp.float32(MAX_NORM) * y, jnp.float32(1.0))
        for r in range(nrows):
            row = g0 + r
            s = sel[r]
            for j in range(8):
                rows[row, pl.ds(16 * j, 16)] = rows[row, pl.ds(16 * j, 16)] * s


def _compute_chunk(rows, nrm_v, lane):
    def group_body(g, _):
        _norm_group(rows, nrm_v, lane, g * 16, 16)
        return _

    lax.fori_loop(0, NFULL, group_body, None)
    _norm_group(rows, nrm_v, lane, NFULL * 16, NTAIL)


def _body(idx_hbm, table_hbm, out_hbm, idx2_v, idx_v, b0, b1, b2, b3,
          g0, g1, g2, g3, w0, w1, w2, w3, nrm_v):
    bufs = (b0, b1, b2, b3)
    gsem = (g0, g1, g2, g3)
    wsem = (w0, w1, w2, w3)
    wid = lax.axis_index("s") * 2 + lax.axis_index("c")
    row_base = wid * ROWS_PER_W
    lane = lax.iota(jnp.int32, 16)

    def stripe_body(s, _):
        pltpu.sync_copy(idx_hbm.at[pl.ds(row_base + s * 64, 64)], idx2_v)

        def build_body(t, _):
            lt = t * 16 + lane
            r = lt // N_COLS
            c = lt - r * N_COLS
            vals = plsc.load_gather(idx2_v, [r, c])
            p = s * (64 * N_COLS) + lt
            q = p // CHUNK
            d = p - q * CHUNK
            plsc.store_scatter(idx_v, [q, d], vals)
            return _

        lax.fori_loop(0, 64 * N_COLS // 16, build_body, None)
        return _

    lax.fori_loop(0, ROWS_PER_W // 64, stripe_body, None)

    pltpu.async_copy(table_hbm.at[idx_v.at[0]], b0, g0)
    pltpu.async_copy(table_hbm.at[idx_v.at[1]], b1, g1)

    def quad_body(i, _):
        for b in range(NBUF):
            c = i * NBUF + b
            bn = (b + 2) % NBUF
            orow = row_base + 2 * c
            pltpu.make_async_copy(table_hbm.at[idx_v.at[c]], bufs[b],
                                  gsem[b]).wait()
            _compute_chunk(bufs[b], nrm_v, lane)

            @pl.when(c >= 2)
            def _retire():
                for _h in range(2):
                    pltpu.make_async_copy(bufs[bn].at[pl.ds(0, N_COLS)],
                                          out_hbm.at[0], wsem[bn]).wait()

            @pl.when(c + 2 < N_CHUNKS)
            def _prefetch():
                pltpu.async_copy(table_hbm.at[idx_v.at[c + 2]], bufs[bn],
                                 gsem[bn])

            pltpu.async_copy(bufs[b].at[pl.ds(0, N_COLS)],
                             out_hbm.at[orow], wsem[b])
            pltpu.async_copy(bufs[b].at[pl.ds(N_COLS, N_COLS)],
                             out_hbm.at[orow + 1], wsem[b])
        return _

    lax.fori_loop(0, N_CHUNKS // NBUF, quad_body, None)

    for b in ((N_CHUNKS - 2) % NBUF, (N_CHUNKS - 1) % NBUF):
        for _h in range(2):
            pltpu.make_async_copy(bufs[b].at[pl.ds(0, N_COLS)],
                                  out_hbm.at[0], wsem[b]).wait()


@jax.jit
def _impl(idx, embeddings):
    mesh = plsc.VectorSubcoreMesh(core_axis_name="c", subcore_axis_name="s")
    f = pl.kernel(
        _body,
        mesh=mesh,
        out_type=jax.ShapeDtypeStruct((N_ROWS, N_COLS, DIM), jnp.float32),
        scratch_types=[
            pltpu.VMEM((64, N_COLS), jnp.int32),
            pltpu.VMEM((N_CHUNKS, CHUNK), jnp.int32),
            pltpu.VMEM((CHUNK, DIM), jnp.float32),
            pltpu.VMEM((CHUNK, DIM), jnp.float32),
            pltpu.VMEM((CHUNK, DIM), jnp.float32),
            pltpu.VMEM((CHUNK, DIM), jnp.float32),
            pltpu.SemaphoreType.DMA,
            pltpu.SemaphoreType.DMA,
            pltpu.SemaphoreType.DMA,
            pltpu.SemaphoreType.DMA,
            pltpu.SemaphoreType.DMA,
            pltpu.SemaphoreType.DMA,
            pltpu.SemaphoreType.DMA,
            pltpu.SemaphoreType.DMA,
            pltpu.VMEM((16, 16), jnp.float32),
        ],
        compiler_params=pltpu.CompilerParams(needs_layout_passes=False),
    )
    return f(idx, embeddings)


def kernel(indices, embeddings):
    return _impl(indices.astype(jnp.int32), embeddings)


# speedup vs baseline: 4.3528x; 1.5238x over previous
import jax
import jax.numpy as jnp
from jax import lax
from jax.experimental import pallas as pl
from jax.experimental.pallas import tpu as pltpu
from jax.experimental.pallas import tpu_sc as plsc

NUM_NODES = 1000000
DIM = 128
EPS = 1e-05
N_ROWS = 16384
N_COLS = 50
NW = 32
ROWS_PER_W = N_ROWS // NW
CHUNK = 128
BLOCKS = ROWS_PER_W // CHUNK
N_CHUNKS = N_COLS * BLOCKS
NBUF = 4
STRIPE = 64

MAX_NORM = 1.0 - EPS
MAX_NORM2 = MAX_NORM * MAX_NORM


def _rsqrt(x):
    bits = lax.bitcast_convert_type(x, jnp.int32)
    y = lax.bitcast_convert_type(
        jnp.int32(0x5F3759DF) - lax.shift_right_arithmetic(bits, 1),
        jnp.float32)
    for _ in range(3):
        y = y * (jnp.float32(1.5) - jnp.float32(0.5) * x * y * y)
    return y


def _compute_chunk(rows, nrm_v, lane):
    def group_body(g, _):
        for r in range(16):
            row = g * 16 + r
            vs = [rows[row, pl.ds(16 * j, 16)] for j in range(8)]
            sq = [v * v for v in vs]
            s0 = (sq[0] + sq[1]) + (sq[2] + sq[3])
            s1 = (sq[4] + sq[5]) + (sq[6] + sq[7])
            nrm_v[r, :] = s0 + s1
        tot = plsc.load_gather(nrm_v, [lane, jnp.full((16,), 0, jnp.int32)])
        for l in range(1, 16):
            tot = tot + plsc.load_gather(
                nrm_v, [lane, jnp.full((16,), l, jnp.int32)])
        over = tot > jnp.float32(MAX_NORM2)

        @pl.when(jnp.any(over))
        def _rescale():
            y = _rsqrt(tot)
            sel = jnp.where(over, jnp.float32(MAX_NORM) * y, jnp.float32(1.0))
            for r in range(16):
                row = g * 16 + r
                s = sel[r]
                for j in range(8):
                    rows[row, pl.ds(16 * j, 16)] = (
                        rows[row, pl.ds(16 * j, 16)] * s)

        return _

    lax.fori_loop(0, CHUNK // 16, group_body, None)


def _body(idx_hbm, table_hbm, out_hbm, idx2_v, idx_v, b0, b1, b2, b3,
          g0, g1, g2, g3, w0, w1, w2, w3, nrm_v):
    bufs = (b0, b1, b2, b3)
    gsem = (g0, g1, g2, g3)
    wsem = (w0, w1, w2, w3)
    wid = lax.axis_index("s") * 2 + lax.axis_index("c")
    row_base = wid * ROWS_PER_W
    lane = lax.iota(jnp.int32, 16)

    def stripe_body(st, _):
        pltpu.sync_copy(idx_hbm.at[pl.ds(row_base + st * STRIPE, STRIPE)],
                        idx2_v)

        def build_body(t, _):
            lt = t * 16 + lane
            lr = lt // N_COLS
            c = lt - lr * N_COLS
            vals = plsc.load_gather(idx2_v, [lr, c])
            r = st * STRIPE + lr
            q = c * BLOCKS + r // CHUNK
            d = r - (r // CHUNK) * CHUNK
            plsc.store_scatter(idx_v, [q, d], vals)
            return _

        lax.fori_loop(0, STRIPE * N_COLS // 16, build_body, None)
        return _

    lax.fori_loop(0, ROWS_PER_W // STRIPE, stripe_body, None)

    pltpu.async_copy(table_hbm.at[idx_v.at[0]], b0, g0)
    pltpu.async_copy(table_hbm.at[idx_v.at[1]], b1, g1)

    def quad_body(i, _):
        for b in range(NBUF):
            c = i * NBUF + b
            bn = (b + 2) % NBUF
            col = c // BLOCKS
            orow = row_base + (c - col * BLOCKS) * CHUNK
            pltpu.make_async_copy(table_hbm.at[idx_v.at[c]], bufs[b],
                                  gsem[b]).wait()
            _compute_chunk(bufs[b], nrm_v, lane)

            @pl.when(c >= 2)
            def _retire():
                pltpu.make_async_copy(bufs[bn], out_hbm.at[0, pl.ds(0, CHUNK)],
                                      wsem[bn]).wait()

            @pl.when(c + 2 < N_CHUNKS)
            def _prefetch():
                pltpu.async_copy(table_hbm.at[idx_v.at[c + 2]], bufs[bn],
                                 gsem[bn])

            pltpu.async_copy(bufs[b], out_hbm.at[col, pl.ds(orow, CHUNK)],
                             wsem[b])
        return _

    lax.fori_loop(0, N_CHUNKS // NBUF, quad_body, None)

    for b in ((N_CHUNKS - 2) % NBUF, (N_CHUNKS - 1) % NBUF):
        pltpu.make_async_copy(bufs[b], out_hbm.at[0, pl.ds(0, CHUNK)],
                              wsem[b]).wait()


@jax.jit
def _impl(idx, embeddings):
    mesh = plsc.VectorSubcoreMesh(core_axis_name="c", subcore_axis_name="s")
    f = pl.kernel(
        _body,
        mesh=mesh,
        out_type=jax.ShapeDtypeStruct((N_COLS, N_ROWS, DIM), jnp.float32),
        scratch_types=[
            pltpu.VMEM((STRIPE, N_COLS), jnp.int32),
            pltpu.VMEM((N_CHUNKS, CHUNK), jnp.int32),
            pltpu.VMEM((CHUNK, DIM), jnp.float32),
            pltpu.VMEM((CHUNK, DIM), jnp.float32),
            pltpu.VMEM((CHUNK, DIM), jnp.float32),
            pltpu.VMEM((CHUNK, DIM), jnp.float32),
            pltpu.SemaphoreType.DMA,
            pltpu.SemaphoreType.DMA,
            pltpu.SemaphoreType.DMA,
            pltpu.SemaphoreType.DMA,
            pltpu.SemaphoreType.DMA,
            pltpu.SemaphoreType.DMA,
            pltpu.SemaphoreType.DMA,
            pltpu.SemaphoreType.DMA,
            pltpu.VMEM((16, 16), jnp.float32),
        ],
        compiler_params=pltpu.CompilerParams(needs_layout_passes=False),
    )
    out = f(idx, embeddings)
    return jnp.transpose(out, (1, 0, 2))


def kernel(indices, embeddings):
    return _impl(indices.astype(jnp.int32), embeddings)
